# baseline (speedup 1.0000x reference)
import functools
import math

import jax
import jax.numpy as jnp
from jax.experimental import pallas as pl
from jax.experimental.pallas import tpu as pltpu

_H = 64
_A = 3
_F = 32
_S = 16
_EPS = 1e-5
_FC = 4
_NC = _F // _FC
_BB = 256


def _elu(x):
    return jnp.where(x > 0, x, jnp.exp(x) - 1.0)


def _ln_lanes(y, g, b):
    m = jnp.mean(y, axis=-1, keepdims=True)
    v = jnp.mean((y - m) ** 2, axis=-1, keepdims=True)
    return (y - m) * jax.lax.rsqrt(v + _EPS) * g + b


def _body(state_ref, act_ref,
          wg_w1, wg_b1, wg_w2, wg_b2, wg_g, wg_bl,
          e_all, w1r, b1r, skw, skb, w2big, b2big, mavg, g1, bl1,
          wih, whh, lb, wq, bq, wkv, bkv, seg, wao, bao, l2g, l2b,
          a1, a1b, a2, a2b, a3, a3b, fcw, fcb, s1, s1b, s2, s2b,
          k1c, k1a, k1b, k2w, k2b, k3w, k3b,
          m1c, m1a, m1b, m2w, m2b, m3w, m3b,
          pol_ref, c1_ref, c2_ref, fc_ref, sf_ref):
    dot = functools.partial(jnp.dot, preferred_element_type=jnp.float32)
    s3 = state_ref[...]
    s2d = s3.reshape(_S * _BB, _F)

    h = _elu(dot(s2d, wg_w1[...]) + wg_b1[...])
    h = dot(h, wg_w2[...]) + wg_b2[...]
    y = s2d + h[:, :_F] * jax.nn.sigmoid(h[:, _F:])
    wl = _ln_lanes(y, wg_g[...], wg_bl[...])
    wmax = jnp.max(wl, axis=-1, keepdims=True)
    we = jnp.exp(wl - wmax)
    w = we / jnp.sum(we, axis=-1, keepdims=True)

    wide = _FC * _H
    acc = jnp.zeros((_S * _BB, wide), jnp.float32)
    for c in range(_NC):
        ecol = e_all[:, c * wide:(c + 1) * wide]
        sw = dot(s2d, ecol)
        z = _elu(sw * w1r[...] + b1r[...])
        hh = dot(z, w2big[...]) + b2big[...]
        glu = hh[:, :wide] * jax.nn.sigmoid(hh[:, wide:])
        yy = sw * skw[...] + skb[...] + glu
        d = yy - dot(yy, mavg[...])
        vv = dot(d * d, mavg[...])
        t = dot(w, ecol) * jax.lax.rsqrt(vv + _EPS)
        acc = acc + t * d
    x2d = (acc[:, 0:_H] + acc[:, _H:2 * _H]
           + acc[:, 2 * _H:3 * _H] + acc[:, 3 * _H:4 * _H]) * g1[...] + bl1[...]

    xw = dot(x2d, wih[...]) + lb[...]
    h_t = jnp.zeros((_BB, _H), jnp.float32)
    c_t = jnp.zeros((_BB, _H), jnp.float32)
    kvs = []
    for t in range(_S):
        g = xw[t * _BB:(t + 1) * _BB, :] + dot(h_t, whh[...])
        i_g = jax.nn.sigmoid(g[:, 0:_H])
        f_g = jax.nn.sigmoid(g[:, _H:2 * _H])
        o_g = jax.nn.sigmoid(g[:, 2 * _H:3 * _H])
        g_g = jnp.tanh(g[:, 3 * _H:])
        c_t = f_g * c_t + i_g * g_g
        h_t = o_g * jnp.tanh(c_t)
        kvs.append(dot(h_t, wkv[...]) + bkv[...])

    q = dot(h_t, wq[...]) + bq[...]
    logits = [dot(kv[:, :_H] * q, seg[...]) for kv in kvs]
    mx = functools.reduce(jnp.maximum, logits)
    es = [jnp.exp(l - mx) for l in logits]
    den = functools.reduce(jnp.add, es)
    inv = 1.0 / den
    att = jnp.zeros((_BB, _H), jnp.float32)
    for t in range(_S):
        att = att + (es[t] * inv) * kvs[t][:, _H:]
    ao = dot(att, wao[...]) + bao[...]
    ctx = _ln_lanes(h_t + ao, l2g[...], l2b[...])

    relu = jax.nn.relu
    ph = relu(dot(ctx, a1[...]) + a1b[...])
    ph = relu(dot(ph, a2[...]) + a2b[...])
    pol_ref[...] = dot(ph, a3[...]) + a3b[...]
    fc_ref[...] = dot(ctx, fcw[...]) + fcb[...]
    sh = relu(dot(ctx, s1[...]) + s1b[...])
    sf_ref[...] = jax.nn.sigmoid(dot(sh, s2[...]) + s2b[...])
    act = act_ref[...]
    for wc, wa, b1_, w2_, b2_, w3_, b3_, ref in (
            (k1c, k1a, k1b, k2w, k2b, k3w, k3b, c1_ref),
            (m1c, m1a, m1b, m2w, m2b, m3w, m3b, c2_ref)):
        hh1 = relu(dot(ctx, wc[...]) + dot(act, wa[...]) + b1_[...])
        hh2 = relu(dot(hh1, w2_[...]) + b2_[...])
        ref[...] = jnp.sum(hh2 * w3_[...], axis=-1, keepdims=True) + b3_[...]


def kernel(state, action_onehot, params):
    p = params
    B = state.shape[0]
    f32 = jnp.float32

    def t2(w):
        return w.T.astype(f32)

    def row(b):
        return b.reshape(1, -1).astype(f32)

    wg_w1 = t2(p['wg_fc1_w']); wg_b1 = row(p['wg_fc1_b'])
    wg_w2 = t2(p['wg_fc2_w']); wg_b2 = row(p['wg_fc2_b'])
    wg_g = row(p['wg_ln_g']); wg_bl = row(p['wg_ln_b'])

    e_all = jnp.repeat(jnp.eye(_F, dtype=f32), _H, axis=1)
    w1r = jnp.tile(p['fg_fc1_w'].reshape(1, _H).astype(f32), (1, _FC))
    b1r = jnp.tile(row(p['fg_fc1_b']), (1, _FC))
    skw = jnp.tile(p['fg_skip_w'].reshape(1, _H).astype(f32), (1, _FC))
    skb = jnp.tile(row(p['fg_skip_b']), (1, _FC))
    w2full = t2(p['fg_fc2_w'])
    ey4 = jnp.eye(_FC, dtype=f32)
    w2big = jnp.concatenate([jnp.kron(ey4, w2full[:, :_H]),
                             jnp.kron(ey4, w2full[:, _H:])], axis=1)
    b2big = jnp.concatenate([jnp.tile(row(p['fg_fc2_b'][:_H]), (1, _FC)),
                             jnp.tile(row(p['fg_fc2_b'][_H:]), (1, _FC))], axis=1)
    mavg = jnp.kron(ey4, jnp.full((_H, _H), 1.0 / _H, f32))
    g1 = row(p['fg_ln_g'])
    bl1 = row(p['fg_ln_b'])

    perm = jnp.concatenate([jnp.arange(0, 2 * _H), jnp.arange(3 * _H, 4 * _H),
                            jnp.arange(2 * _H, 3 * _H)])
    wih = p['lstm_wih'][perm].T.astype(f32)
    whh = p['lstm_whh'][perm].T.astype(f32)
    lb = (p['lstm_bih'] + p['lstm_bhh'])[perm].reshape(1, -1).astype(f32)

    wq = t2(p['attn_in_w'][:_H]); bq = row(p['attn_in_b'][:_H])
    wkv = t2(p['attn_in_w'][_H:]); bkv = row(p['attn_in_b'][_H:])
    hd = _H // 2
    seg = jnp.kron(jnp.eye(2, dtype=f32),
                   jnp.full((hd, hd), 1.0 / math.sqrt(hd), f32))
    wao = t2(p['attn_out_w']); bao = row(p['attn_out_b'])
    l2g = row(p['ln2_g']); l2b = row(p['ln2_b'])

    a1 = t2(p['a1_w']); a1b = row(p['a1_b'])
    a2 = t2(p['a2_w']); a2b = row(p['a2_b'])
    a3 = t2(p['a3_w']); a3b = row(p['a3_b'])
    fcw = t2(p['fc_w']); fcb = row(p['fc_b'])
    s1 = t2(p['s1_w']); s1b = row(p['s1_b'])
    s2 = t2(p['s2_w']); s2b = row(p['s2_b'])
    crit = []
    for pre in ('c1', 'c2'):
        crit += [p[pre + '_1w'][:, :_H].T.astype(f32),
                 p[pre + '_1w'][:, _H:].T.astype(f32),
                 row(p[pre + '_1b']),
                 t2(p[pre + '_2w']), row(p[pre + '_2b']),
                 p[pre + '_3w'].reshape(1, _H).astype(f32),
                 p[pre + '_3b'].reshape(1, 1).astype(f32)]

    weights = [wg_w1, wg_b1, wg_w2, wg_b2, wg_g, wg_bl,
               e_all, w1r, b1r, skw, skb, w2big, b2big, mavg, g1, bl1,
               wih, whh, lb, wq, bq, wkv, bkv, seg, wao, bao, l2g, l2b,
               a1, a1b, a2, a2b, a3, a3b, fcw, fcb, s1, s1b, s2, s2b] + crit

    state_t = state.transpose(1, 0, 2)

    in_specs = [pl.BlockSpec((_S, _BB, _F), lambda i: (0, i, 0)),
                pl.BlockSpec((_BB, _A), lambda i: (i, 0))]
    for wgt in weights:
        nd = wgt.ndim
        in_specs.append(pl.BlockSpec(wgt.shape, lambda i, _n=nd: (0,) * _n))

    out_shape = [jax.ShapeDtypeStruct((B, _A), f32),
                 jax.ShapeDtypeStruct((B, 1), f32),
                 jax.ShapeDtypeStruct((B, 1), f32),
                 jax.ShapeDtypeStruct((B, _F), f32),
                 jax.ShapeDtypeStruct((B, _A), f32)]
    out_specs = [pl.BlockSpec((_BB, _A), lambda i: (i, 0)),
                 pl.BlockSpec((_BB, 1), lambda i: (i, 0)),
                 pl.BlockSpec((_BB, 1), lambda i: (i, 0)),
                 pl.BlockSpec((_BB, _F), lambda i: (i, 0)),
                 pl.BlockSpec((_BB, _A), lambda i: (i, 0))]

    pol, c1, c2, fc, sf = pl.pallas_call(
        _body,
        grid=(B // _BB,),
        in_specs=in_specs,
        out_specs=out_specs,
        out_shape=out_shape,
        compiler_params=pltpu.CompilerParams(
            dimension_semantics=("parallel",),
        ),
        name="tft_actor_critic",
    )(state_t, action_onehot, *weights)
    return pol, c1[:, 0], c2[:, 0], fc, sf


# speedup vs baseline: 1.0881x; 1.0881x over previous
import functools
import math

import jax
import jax.numpy as jnp
from jax.experimental import pallas as pl
from jax.experimental.pallas import tpu as pltpu

_H = 64
_A = 3
_F = 32
_S = 16
_EPS = 1e-5
_FC = 4
_NC = _F // _FC
_BB = 128


def _elu(x):
    return jnp.where(x > 0, x, jnp.exp(x) - 1.0)


def _ln_lanes(y, g, b):
    m = jnp.mean(y, axis=-1, keepdims=True)
    v = jnp.mean((y - m) ** 2, axis=-1, keepdims=True)
    return (y - m) * jax.lax.rsqrt(v + _EPS) * g + b


def _body(state_ref, act_ref,
          wg_w1, wg_b1, wg_w2, wg_b2, wg_g, wg_bl,
          e_all, w1r, b1r, skw, skb, w2big, b2big, mavg, g1, bl1,
          wih, whh, lb, wq, bq, wkv, bkv, seg, wao, bao, l2g, l2b,
          a1, a1b, a2, a2b, a3, a3b, fcw, fcb, s1, s1b, s2, s2b,
          k1c, k1a, k1b, k2w, k2b, k3w, k3b,
          m1c, m1a, m1b, m2w, m2b, m3w, m3b,
          pol_ref, c1_ref, c2_ref, fc_ref, sf_ref):
    dot = functools.partial(jnp.dot, preferred_element_type=jnp.float32)
    s3 = state_ref[...]
    s2d = s3.reshape(_S * _BB, _F)

    h = _elu(dot(s2d, wg_w1[...]) + wg_b1[...])
    h = dot(h, wg_w2[...]) + wg_b2[...]
    y = s2d + h[:, :_F] * jax.nn.sigmoid(h[:, _F:])
    wl = _ln_lanes(y, wg_g[...], wg_bl[...])
    wmax = jnp.max(wl, axis=-1, keepdims=True)
    we = jnp.exp(wl - wmax)
    w = we / jnp.sum(we, axis=-1, keepdims=True)

    wide = _FC * _H
    acc = jnp.zeros((_S * _BB, wide), jnp.float32)
    for c in range(_NC):
        ecol = e_all[:, c * wide:(c + 1) * wide]
        sw = dot(s2d, ecol)
        z = _elu(sw * w1r[...] + b1r[...])
        hh = dot(z, w2big[...]) + b2big[...]
        glu = hh[:, :wide] * jax.nn.sigmoid(hh[:, wide:])
        yy = sw * skw[...] + skb[...] + glu
        d = yy - dot(yy, mavg[...])
        vv = dot(d * d, mavg[...])
        t = dot(w, ecol) * jax.lax.rsqrt(vv + _EPS)
        acc = acc + t * d
    x2d = (acc[:, 0:_H] + acc[:, _H:2 * _H]
           + acc[:, 2 * _H:3 * _H] + acc[:, 3 * _H:4 * _H]) * g1[...] + bl1[...]

    xw = dot(x2d, wih[...]) + lb[...]
    h_t = jnp.zeros((_BB, _H), jnp.float32)
    c_t = jnp.zeros((_BB, _H), jnp.float32)
    kvs = []
    for t in range(_S):
        g = xw[t * _BB:(t + 1) * _BB, :] + dot(h_t, whh[...])
        i_g = jax.nn.sigmoid(g[:, 0:_H])
        f_g = jax.nn.sigmoid(g[:, _H:2 * _H])
        o_g = jax.nn.sigmoid(g[:, 2 * _H:3 * _H])
        g_g = jnp.tanh(g[:, 3 * _H:])
        c_t = f_g * c_t + i_g * g_g
        h_t = o_g * jnp.tanh(c_t)
        kvs.append(dot(h_t, wkv[...]) + bkv[...])

    q = dot(h_t, wq[...]) + bq[...]
    logits = [dot(kv[:, :_H] * q, seg[...]) for kv in kvs]
    mx = functools.reduce(jnp.maximum, logits)
    es = [jnp.exp(l - mx) for l in logits]
    den = functools.reduce(jnp.add, es)
    inv = 1.0 / den
    att = jnp.zeros((_BB, _H), jnp.float32)
    for t in range(_S):
        att = att + (es[t] * inv) * kvs[t][:, _H:]
    ao = dot(att, wao[...]) + bao[...]
    ctx = _ln_lanes(h_t + ao, l2g[...], l2b[...])

    relu = jax.nn.relu
    ph = relu(dot(ctx, a1[...]) + a1b[...])
    ph = relu(dot(ph, a2[...]) + a2b[...])
    pol_ref[...] = dot(ph, a3[...]) + a3b[...]
    fc_ref[...] = dot(ctx, fcw[...]) + fcb[...]
    sh = relu(dot(ctx, s1[...]) + s1b[...])
    sf_ref[...] = jax.nn.sigmoid(dot(sh, s2[...]) + s2b[...])
    act = act_ref[...]
    for wc, wa, b1_, w2_, b2_, w3_, b3_, ref in (
            (k1c, k1a, k1b, k2w, k2b, k3w, k3b, c1_ref),
            (m1c, m1a, m1b, m2w, m2b, m3w, m3b, c2_ref)):
        hh1 = relu(dot(ctx, wc[...]) + dot(act, wa[...]) + b1_[...])
        hh2 = relu(dot(hh1, w2_[...]) + b2_[...])
        ref[...] = jnp.sum(hh2 * w3_[...], axis=-1, keepdims=True) + b3_[...]


def kernel(state, action_onehot, params):
    p = params
    B = state.shape[0]
    f32 = jnp.float32

    def t2(w):
        return w.T.astype(f32)

    def row(b):
        return b.reshape(1, -1).astype(f32)

    wg_w1 = t2(p['wg_fc1_w']); wg_b1 = row(p['wg_fc1_b'])
    wg_w2 = t2(p['wg_fc2_w']); wg_b2 = row(p['wg_fc2_b'])
    wg_g = row(p['wg_ln_g']); wg_bl = row(p['wg_ln_b'])

    e_all = jnp.repeat(jnp.eye(_F, dtype=f32), _H, axis=1)
    w1r = jnp.tile(p['fg_fc1_w'].reshape(1, _H).astype(f32), (1, _FC))
    b1r = jnp.tile(row(p['fg_fc1_b']), (1, _FC))
    skw = jnp.tile(p['fg_skip_w'].reshape(1, _H).astype(f32), (1, _FC))
    skb = jnp.tile(row(p['fg_skip_b']), (1, _FC))
    w2full = t2(p['fg_fc2_w'])
    ey4 = jnp.eye(_FC, dtype=f32)
    w2big = jnp.concatenate([jnp.kron(ey4, w2full[:, :_H]),
                             jnp.kron(ey4, w2full[:, _H:])], axis=1)
    b2big = jnp.concatenate([jnp.tile(row(p['fg_fc2_b'][:_H]), (1, _FC)),
                             jnp.tile(row(p['fg_fc2_b'][_H:]), (1, _FC))], axis=1)
    mavg = jnp.kron(ey4, jnp.full((_H, _H), 1.0 / _H, f32))
    g1 = row(p['fg_ln_g'])
    bl1 = row(p['fg_ln_b'])

    perm = jnp.concatenate([jnp.arange(0, 2 * _H), jnp.arange(3 * _H, 4 * _H),
                            jnp.arange(2 * _H, 3 * _H)])
    wih = p['lstm_wih'][perm].T.astype(f32)
    whh = p['lstm_whh'][perm].T.astype(f32)
    lb = (p['lstm_bih'] + p['lstm_bhh'])[perm].reshape(1, -1).astype(f32)

    wq = t2(p['attn_in_w'][:_H]); bq = row(p['attn_in_b'][:_H])
    wkv = t2(p['attn_in_w'][_H:]); bkv = row(p['attn_in_b'][_H:])
    hd = _H // 2
    seg = jnp.kron(jnp.eye(2, dtype=f32),
                   jnp.full((hd, hd), 1.0 / math.sqrt(hd), f32))
    wao = t2(p['attn_out_w']); bao = row(p['attn_out_b'])
    l2g = row(p['ln2_g']); l2b = row(p['ln2_b'])

    a1 = t2(p['a1_w']); a1b = row(p['a1_b'])
    a2 = t2(p['a2_w']); a2b = row(p['a2_b'])
    a3 = t2(p['a3_w']); a3b = row(p['a3_b'])
    fcw = t2(p['fc_w']); fcb = row(p['fc_b'])
    s1 = t2(p['s1_w']); s1b = row(p['s1_b'])
    s2 = t2(p['s2_w']); s2b = row(p['s2_b'])
    crit = []
    for pre in ('c1', 'c2'):
        crit += [p[pre + '_1w'][:, :_H].T.astype(f32),
                 p[pre + '_1w'][:, _H:].T.astype(f32),
                 row(p[pre + '_1b']),
                 t2(p[pre + '_2w']), row(p[pre + '_2b']),
                 p[pre + '_3w'].reshape(1, _H).astype(f32),
                 p[pre + '_3b'].reshape(1, 1).astype(f32)]

    weights = [wg_w1, wg_b1, wg_w2, wg_b2, wg_g, wg_bl,
               e_all, w1r, b1r, skw, skb, w2big, b2big, mavg, g1, bl1,
               wih, whh, lb, wq, bq, wkv, bkv, seg, wao, bao, l2g, l2b,
               a1, a1b, a2, a2b, a3, a3b, fcw, fcb, s1, s1b, s2, s2b] + crit

    state_t = state.transpose(1, 0, 2)

    in_specs = [pl.BlockSpec((_S, _BB, _F), lambda i: (0, i, 0)),
                pl.BlockSpec((_BB, _A), lambda i: (i, 0))]
    for wgt in weights:
        nd = wgt.ndim
        in_specs.append(pl.BlockSpec(wgt.shape, lambda i, _n=nd: (0,) * _n))

    out_shape = [jax.ShapeDtypeStruct((B, _A), f32),
                 jax.ShapeDtypeStruct((B, 1), f32),
                 jax.ShapeDtypeStruct((B, 1), f32),
                 jax.ShapeDtypeStruct((B, _F), f32),
                 jax.ShapeDtypeStruct((B, _A), f32)]
    out_specs = [pl.BlockSpec((_BB, _A), lambda i: (i, 0)),
                 pl.BlockSpec((_BB, 1), lambda i: (i, 0)),
                 pl.BlockSpec((_BB, 1), lambda i: (i, 0)),
                 pl.BlockSpec((_BB, _F), lambda i: (i, 0)),
                 pl.BlockSpec((_BB, _A), lambda i: (i, 0))]

    pol, c1, c2, fc, sf = pl.pallas_call(
        _body,
        grid=(B // _BB,),
        in_specs=in_specs,
        out_specs=out_specs,
        out_shape=out_shape,
        compiler_params=pltpu.CompilerParams(
            dimension_semantics=("parallel",),
        ),
        name="tft_actor_critic",
    )(state_t, action_onehot, *weights)
    return pol, c1[:, 0], c2[:, 0], fc, sf
